# baseline (speedup 1.0000x reference)
import jax
import jax.numpy as jnp
from jax.experimental import pallas as pl
from jax.experimental.pallas import tpu as pltpu
from jax.experimental.pallas import tpu_sc as plsc

_EMB = 32
_SCALE = float(_EMB) ** 0.5
_LANES = 16
_WINDOW = 1024


def _gather_scale(xt, table):
    n_cols, n_rows = xt.shape
    mesh = plsc.VectorSubcoreMesh(core_axis_name="c", subcore_axis_name="s")

    @pl.kernel(
        out_type=jax.ShapeDtypeStruct((n_cols, n_rows, _EMB), jnp.float32),
        mesh=mesh,
        scratch_types=[pltpu.VMEM((_WINDOW,), jnp.int32)],
        compiler_params=pltpu.CompilerParams(
            use_tc_tiling_on_sc=False, needs_layout_passes=False
        ),
    )
    def k(xt_hbm, table_hbm, out_hbm, idx_i32):
        def body(idx_vmem, out_vmem):
            idx_f32 = idx_vmem.at[0]

            @plsc.parallel_loop(0, _WINDOW, step=_LANES, unroll=4)
            def _(j):
                sl = pl.ds(j, _LANES)
                idx_i32.at[sl][...] = plsc.bitcast(
                    idx_f32.at[sl][...], jnp.int32
                )

            rows = out_vmem.at[0]
            pltpu.sync_copy(table_hbm.at[idx_i32], rows)

            @plsc.parallel_loop(0, _WINDOW, unroll=8)
            def _(j):
                for c in range(_EMB // _LANES):
                    sl = (j, pl.ds(c * _LANES, _LANES))
                    rows.at[sl][...] = rows.at[sl][...] * _SCALE

        pltpu.emit_pipeline(
            body,
            grid=(n_cols, n_rows // _WINDOW),
            in_specs=[pl.BlockSpec((1, _WINDOW), lambda j, i: (j, i))],
            out_specs=[pl.BlockSpec((1, _WINDOW, _EMB), lambda j, i: (j, i, 0))],
            core_axis_name=("c", "s"),
            dimension_semantics=(pltpu.PARALLEL, pltpu.PARALLEL),
        )(xt_hbm, out_hbm)

    return k(xt, table)


def kernel(x, table):
    if x.dtype != jnp.int32:
        x = x.astype(jnp.int32)
    xf = jax.lax.bitcast_convert_type(x, jnp.float32)
    out_t = _gather_scale(xf.T, table)
    return jnp.transpose(out_t, (1, 0, 2))


# speedup vs baseline: 1.0029x; 1.0029x over previous
import jax
import jax.numpy as jnp
from jax.experimental import pallas as pl
from jax.experimental.pallas import tpu as pltpu
from jax.experimental.pallas import tpu_sc as plsc

_EMB = 32
_SCALE = float(_EMB) ** 0.5
_LANES = 16
_WINDOW = 1024
_TC_BLOCK_ROWS = 8


def _tc_flatten(xt):
    n_cols, n_rows = xt.shape
    blk = _TC_BLOCK_ROWS * n_rows

    def body(x_ref, o_ref):
        o_ref[...] = x_ref[...].reshape(blk)

    return pl.pallas_call(
        body,
        grid=(n_cols // _TC_BLOCK_ROWS,),
        in_specs=[pl.BlockSpec((_TC_BLOCK_ROWS, n_rows), lambda a: (a, 0))],
        out_specs=pl.BlockSpec((blk,), lambda a: (a,)),
        out_shape=jax.ShapeDtypeStruct((n_cols * n_rows,), jnp.int32),
    )(xt)


def _gather_scale(idx2d, table):
    n_cols, n_rows = idx2d.shape
    mesh = plsc.VectorSubcoreMesh(core_axis_name="c", subcore_axis_name="s")

    @pl.kernel(
        out_type=jax.ShapeDtypeStruct((n_cols, n_rows, _EMB), jnp.float32),
        mesh=mesh,
        compiler_params=pltpu.CompilerParams(use_tc_tiling_on_sc=False),
    )
    def k(idx_hbm, table_hbm, out_hbm):
        def body(idx_vmem, out_vmem):
            rows = out_vmem.at[0]
            pltpu.sync_copy(table_hbm.at[idx_vmem.at[0]], rows)

            @plsc.parallel_loop(0, _WINDOW, unroll=8)
            def _(j):
                for c in range(_EMB // _LANES):
                    sl = (pl.ds(j, 1), pl.ds(c * _LANES, _LANES))
                    rows.at[sl][...] = rows.at[sl][...] * _SCALE

        pltpu.emit_pipeline(
            body,
            grid=(n_cols, n_rows // _WINDOW),
            in_specs=[pl.BlockSpec((1, _WINDOW), lambda j, i: (j, i))],
            out_specs=[pl.BlockSpec((1, _WINDOW, _EMB), lambda j, i: (j, i, 0))],
            core_axis_name=("c", "s"),
            dimension_semantics=(pltpu.PARALLEL, pltpu.PARALLEL),
        )(idx_hbm, out_hbm)

    return k(idx2d, table)


def kernel(x, table):
    if x.dtype != jnp.int32:
        x = x.astype(jnp.int32)
    n_cols, n_rows = x.shape[1], x.shape[0]
    idx_flat = _tc_flatten(x.T)
    out_t = _gather_scale(idx_flat.reshape(n_cols, n_rows), table)
    return jnp.transpose(out_t, (1, 0, 2))
